# Initial kernel scaffold; baseline (speedup 1.0000x reference)
import functools

import jax
import jax.numpy as jnp
from jax import lax
from jax.experimental import pallas as pl
from jax.experimental.pallas import tpu as pltpu
from jax.experimental.pallas import tpu_sc as plsc

N = 10000
E = 320000
D = 128

NC = 2
NS = 16
NW = NC * NS
E_PER_TILE = E // NW
CHUNK = 80
NCHUNK = E_PER_TILE // CHUNK
ROWS_PER_TILE = N // NS


def _sc_aggregate_body(x_hbm, col_hbm, row_hbm, val_hbm, zero_hbm, out_hbm,
                       col_v, row_v, val_v, rows, acc, sem):
    c = lax.axis_index("c")
    s = lax.axis_index("s")
    wid = c * NS + s

    pltpu.sync_copy(col_hbm.at[wid], col_v)
    pltpu.sync_copy(row_hbm.at[wid], row_v)
    pltpu.sync_copy(val_hbm.at[wid], val_v)

    pltpu.sync_copy(zero_hbm, acc.at[pl.ds(s * ROWS_PER_TILE, ROWS_PER_TILE)])
    plsc.subcore_barrier()

    def chunk_body(ci, carry):
        pltpu.async_copy(x_hbm.at[col_v.at[ci]], rows, sem).wait()

        def edge_body(e, carry2):
            v = val_v[ci, e]
            for j in range(D // 16):
                sl = pl.ds(16 * j, 16)
                rows[e, sl] = rows[e, sl] * v
            return carry2
        lax.fori_loop(0, CHUNK, edge_body, 0, unroll=False)

        pltpu.sync_copy(rows, acc.at[row_v.at[ci]], add=True)
        return carry
    lax.fori_loop(0, NCHUNK, chunk_body, 0, unroll=False)

    plsc.subcore_barrier()
    sl = pl.ds(s * ROWS_PER_TILE, ROWS_PER_TILE)
    pltpu.sync_copy(acc.at[sl], out_hbm.at[c, sl])


_sc_aggregate = functools.partial(
    pl.kernel,
    out_type=jax.ShapeDtypeStruct((NC, N, D), jnp.float32),
    mesh=plsc.VectorSubcoreMesh(
        core_axis_name="c", subcore_axis_name="s", num_cores=NC,
        num_subcores=NS),
    scratch_types=[
        pltpu.VMEM((NCHUNK, CHUNK), jnp.int32),
        pltpu.VMEM((NCHUNK, CHUNK), jnp.int32),
        pltpu.VMEM((NCHUNK, CHUNK), jnp.float32),
        pltpu.VMEM((CHUNK, D), jnp.float32),
        pltpu.VMEM_SHARED((N, D), jnp.float32),
        pltpu.SemaphoreType.DMA,
    ],
)(_sc_aggregate_body)


def _tc_matmul_body(p_ref, th_ref, b_ref, o_ref):
    mx = p_ref[0] + p_ref[1]
    o = jnp.dot(mx, th_ref[...], preferred_element_type=jnp.float32)
    o_ref[...] = jnp.maximum(o + b_ref[...], 0.0)


def _tc_matmul(partials, theta, bias):
    blk = 1000
    return pl.pallas_call(
        _tc_matmul_body,
        grid=(N // blk,),
        in_specs=[
            pl.BlockSpec((NC, blk, D), lambda i: (0, i, 0)),
            pl.BlockSpec((D, D), lambda i: (0, 0)),
            pl.BlockSpec((1, D), lambda i: (0, 0)),
        ],
        out_specs=pl.BlockSpec((blk, D), lambda i: (i, 0)),
        out_shape=jax.ShapeDtypeStruct((N, D), jnp.float32),
    )(partials, theta, bias.reshape(1, D))


def kernel(x, edge_val, theta, bias, edge_row, edge_col):
    col3 = edge_col.reshape(NW, NCHUNK, CHUNK)
    row3 = edge_row.reshape(NW, NCHUNK, CHUNK)
    val3 = edge_val.reshape(NW, NCHUNK, CHUNK)
    zero = jnp.zeros((ROWS_PER_TILE, D), jnp.float32)
    partials = _sc_aggregate(x, col3, row3, val3, zero)
    return _tc_matmul(partials, theta, bias)


# speedup vs baseline: 5.1989x; 5.1989x over previous
import functools

import jax
import jax.numpy as jnp
from jax import lax
from jax.experimental import pallas as pl
from jax.experimental.pallas import tpu as pltpu
from jax.experimental.pallas import tpu_sc as plsc

N = 10000
E = 320000
D = 128

NC = 2
NS = 16
NW = NC * NS
CHUNK = 128
NCHUNK = 79
E_PER_TILE = NCHUNK * CHUNK
EPAD = NW * E_PER_TILE
NPAD = 10240
ROWS_PER_TILE = NPAD // NS


def _sc_aggregate_body(x_hbm, col_hbm, row_hbm, val_hbm, zero_hbm, out_hbm,
                       col_v, row_v, val_v, rows, acc, sem):
    c = lax.axis_index("c")
    s = lax.axis_index("s")
    wid = c * NS + s

    pltpu.sync_copy(col_hbm.at[wid], col_v)
    pltpu.sync_copy(row_hbm.at[wid], row_v)
    pltpu.sync_copy(val_hbm.at[wid], val_v)

    pltpu.sync_copy(zero_hbm, acc.at[pl.ds(s * ROWS_PER_TILE, ROWS_PER_TILE)])
    plsc.subcore_barrier()

    def chunk_body(ci, carry):
        pltpu.async_copy(x_hbm.at[col_v.at[ci]], rows, sem).wait()

        def group_body(g, carry2):
            vv = val_v[ci, pl.ds(16 * g, 16)]
            base = 16 * g
            for i in range(16):
                v = vv[i]
                for j in range(D // 16):
                    sl = pl.ds(16 * j, 16)
                    rows[base + i, sl] = rows[base + i, sl] * v
            return carry2
        lax.fori_loop(0, CHUNK // 16, group_body, 0, unroll=False)

        pltpu.sync_copy(rows, acc.at[row_v.at[ci]], add=True)
        return carry
    lax.fori_loop(0, NCHUNK, chunk_body, 0, unroll=False)

    plsc.subcore_barrier()
    sl = pl.ds(s * ROWS_PER_TILE, ROWS_PER_TILE)
    pltpu.sync_copy(acc.at[sl], out_hbm.at[c, sl])


_sc_aggregate = functools.partial(
    pl.kernel,
    out_type=jax.ShapeDtypeStruct((NC, NPAD, D), jnp.float32),
    mesh=plsc.VectorSubcoreMesh(
        core_axis_name="c", subcore_axis_name="s", num_cores=NC,
        num_subcores=NS),
    scratch_types=[
        pltpu.VMEM((NCHUNK, CHUNK), jnp.int32),
        pltpu.VMEM((NCHUNK, CHUNK), jnp.int32),
        pltpu.VMEM((NCHUNK, CHUNK), jnp.float32),
        pltpu.VMEM((CHUNK, D), jnp.float32),
        pltpu.VMEM_SHARED((NPAD, D), jnp.float32),
        pltpu.SemaphoreType.DMA,
    ],
)(_sc_aggregate_body)


def _tc_matmul_body(p_ref, th_ref, b_ref, o_ref):
    mx = p_ref[0] + p_ref[1]
    o = jnp.dot(mx, th_ref[...], preferred_element_type=jnp.float32)
    o_ref[...] = jnp.maximum(o + b_ref[...], 0.0)


def _tc_matmul(partials, theta, bias):
    blk = 1000
    return pl.pallas_call(
        _tc_matmul_body,
        grid=(N // blk,),
        in_specs=[
            pl.BlockSpec((NC, blk, D), lambda i: (0, i, 0)),
            pl.BlockSpec((D, D), lambda i: (0, 0)),
            pl.BlockSpec((1, D), lambda i: (0, 0)),
        ],
        out_specs=pl.BlockSpec((blk, D), lambda i: (i, 0)),
        out_shape=jax.ShapeDtypeStruct((N, D), jnp.float32),
    )(partials, theta, bias.reshape(1, D))


def kernel(x, edge_val, theta, bias, edge_row, edge_col):
    npad = EPAD - E
    col3 = jnp.concatenate(
        [edge_col, jnp.zeros((npad,), jnp.int32)]).reshape(NW, NCHUNK, CHUNK)
    row3 = jnp.concatenate(
        [edge_row, jnp.zeros((npad,), jnp.int32)]).reshape(NW, NCHUNK, CHUNK)
    val3 = jnp.concatenate(
        [edge_val, jnp.zeros((npad,), jnp.float32)]).reshape(NW, NCHUNK, CHUNK)
    zero = jnp.zeros((ROWS_PER_TILE, D), jnp.float32)
    partials = _sc_aggregate(x, col3, row3, val3, zero)
    return _tc_matmul(partials, theta, bias)
